# baseline (speedup 1.0000x reference)
import functools

import jax
import jax.numpy as jnp
from jax import lax
from jax.experimental import pallas as pl
from jax.experimental.pallas import tpu as pltpu
from jax.experimental.pallas import tpu_sc as plsc

B = 16384
K = 32
TPR = 8
FIRE = 16


@functools.cache
def _sc_gather4():
    info = plsc.get_sparse_core_info()
    nc, ns = info.num_cores, info.num_subcores
    nw = nc * ns
    bpw = B // nw
    mesh = plsc.VectorSubcoreMesh(core_axis_name="c", subcore_axis_name="s")

    @functools.partial(
        pl.kernel,
        mesh=mesh,
        out_type=[jax.ShapeDtypeStruct((B, K), jnp.float32)] * 4,
        scratch_types=[
            pltpu.VMEM((bpw,), jnp.int32),
            pltpu.VMEM((bpw,), jnp.int32),
            pltpu.VMEM((bpw,), jnp.int32),
            pltpu.VMEM((bpw,), jnp.int32),
            pltpu.VMEM((bpw, K), jnp.float32),
            pltpu.SemaphoreType.DMA,
        ],
        compiler_params=pltpu.CompilerParams(needs_layout_passes=False),
    )
    def gather4(u3, i3, t3, uix, iix, pix, nix,
                out_u, out_i, out_p, out_n,
                iv0, iv1, iv2, iv3, stage, sem):
        wid = lax.axis_index("s") * nc + lax.axis_index("c")
        sl = pl.ds(wid * bpw, bpw)
        pltpu.sync_copy(uix.at[sl], iv0)
        pltpu.sync_copy(iix.at[sl], iv1)
        pltpu.sync_copy(pix.at[sl], iv2)
        pltpu.sync_copy(nix.at[sl], iv3)

        for src, ivr, outr in ((u3, iv0, out_u), (i3, iv1, out_i),
                               (t3, iv2, out_p), (t3, iv3, out_n)):
            def fire(kblk, carry, src=src, ivr=ivr):
                kb = kblk * FIRE
                idxv = ivr[pl.ds(kb, FIRE)]
                gvv = lax.shift_right_logical(idxv, 3)
                svv = lax.bitwise_and(idxv, TPR - 1)
                for j in range(FIRE):
                    pltpu.async_copy(src.at[gvv[j], pl.ds(svv[j], 1)],
                                     stage.at[pl.ds(kb + j, 1)], sem)
                return carry

            lax.fori_loop(0, bpw // FIRE, fire, 0)
            pltpu.make_async_copy(outr.at[sl], stage, sem).wait()
            pltpu.sync_copy(stage, outr.at[sl])

    return gather4


def _tc_body(u_ref, i_ref, p_ref, n_ref, wu_ref, wi_ref, bu_ref, bi_ref, o_ref):
    t = p_ref[...]
    nt = n_ref[...]
    wu = wu_ref[...]
    wi = wi_ref[...]
    bu = bu_ref[...]
    bi = bi_ref[...]
    dot = functools.partial(jnp.dot, preferred_element_type=jnp.float32)
    su = jax.nn.sigmoid(dot(t, wu) + bu)
    si = jax.nn.sigmoid(dot(t, wi) + bi)
    snu = jax.nn.sigmoid(dot(nt, wu) + bu)
    sni = jax.nn.sigmoid(dot(nt, wi) + bi)
    o_ref[...] = jnp.sum(u_ref[...] * (su - snu) + i_ref[...] * (si - sni), axis=1)


def _tc_dense(u, i, p, n, wut, wit, bu2, bi2):
    blk = 8192
    row_spec = pl.BlockSpec((blk, K), lambda b: (b, 0))
    full_spec = pl.BlockSpec((K, K), lambda b: (0, 0))
    bias_spec = pl.BlockSpec((1, K), lambda b: (0, 0))
    return pl.pallas_call(
        _tc_body,
        grid=(B // blk,),
        in_specs=[row_spec, row_spec, row_spec, row_spec,
                  full_spec, full_spec, bias_spec, bias_spec],
        out_specs=pl.BlockSpec((blk,), lambda b: (b,)),
        out_shape=jax.ShapeDtypeStruct((B,), jnp.float32),
    )(u, i, p, n, wut, wit, bu2, bi2)


def kernel(x, userVecs, itemVecs, tagVecs, Wu, bu, Wi, bi):
    x32 = x.astype(jnp.int32)
    uidx = x32[:, 0]
    iidx = x32[:, 1]
    pidx = x32[:, 2]
    nidx = x32[:, 3]
    u3 = userVecs.reshape(-1, TPR, K)
    i3 = itemVecs.reshape(-1, TPR, K)
    t3 = tagVecs.reshape(-1, TPR, K)
    gu, gi, gp, gn = _sc_gather4()(u3, i3, t3, uidx, iidx, pidx, nidx)
    return _tc_dense(gu, gi, gp, gn, Wu.T, Wi.T,
                     bu.reshape(1, K), bi.reshape(1, K))


# speedup vs baseline: 1.0057x; 1.0057x over previous
import functools

import jax
import jax.numpy as jnp
from jax import lax
from jax.experimental import pallas as pl
from jax.experimental.pallas import tpu as pltpu
from jax.experimental.pallas import tpu_sc as plsc

B = 16384
K = 32
TPR = 8
FIRE = 16


@functools.cache
def _sc_gather4():
    info = plsc.get_sparse_core_info()
    nc, ns = info.num_cores, info.num_subcores
    nw = nc * ns
    bpw = B // nw
    mesh = plsc.VectorSubcoreMesh(core_axis_name="c", subcore_axis_name="s")

    @functools.partial(
        pl.kernel,
        mesh=mesh,
        out_type=[jax.ShapeDtypeStruct((B, K), jnp.float32)] * 4,
        scratch_types=[
            pltpu.VMEM((bpw,), jnp.int32),
            pltpu.VMEM((bpw,), jnp.int32),
            pltpu.VMEM((bpw,), jnp.int32),
            pltpu.VMEM((bpw,), jnp.int32),
            pltpu.VMEM((bpw, K), jnp.float32),
            pltpu.SemaphoreType.DMA,
        ],
        compiler_params=pltpu.CompilerParams(needs_layout_passes=False),
    )
    def gather4(u3, i3, t3, uix, iix, pix, nix,
                out_u, out_i, out_p, out_n,
                iv0, iv1, iv2, iv3, stage, sem):
        wid = lax.axis_index("s") * nc + lax.axis_index("c")
        sl = pl.ds(wid * bpw, bpw)
        pltpu.sync_copy(uix.at[sl], iv0)
        pltpu.sync_copy(iix.at[sl], iv1)
        pltpu.sync_copy(pix.at[sl], iv2)
        pltpu.sync_copy(nix.at[sl], iv3)

        for src, ivr, outr in ((u3, iv0, out_u), (i3, iv1, out_i),
                               (t3, iv2, out_p), (t3, iv3, out_n)):
            def fire(kblk, carry, src=src, ivr=ivr):
                kb = kblk * FIRE
                idxv = ivr[pl.ds(kb, FIRE)]
                gvv = lax.shift_right_logical(idxv, 3)
                svv = lax.bitwise_and(idxv, TPR - 1)
                for j in range(FIRE):
                    pltpu.async_copy(src.at[gvv[j], pl.ds(svv[j], 1)],
                                     stage.at[pl.ds(kb + j, 1)], sem)
                return carry

            lax.fori_loop(0, bpw // FIRE, fire, 0)
            pltpu.make_async_copy(outr.at[sl], stage, sem).wait()
            pltpu.sync_copy(stage, outr.at[sl])

    return gather4


def _tc_body(u_ref, i_ref, p_ref, n_ref, wu_ref, wi_ref, bu_ref, bi_ref, o_ref):
    t = p_ref[...]
    nt = n_ref[...]
    wu = wu_ref[...]
    wi = wi_ref[...]
    bu = bu_ref[...]
    bi = bi_ref[...]
    dot = functools.partial(jnp.dot, preferred_element_type=jnp.float32)
    su = jax.nn.sigmoid(dot(t, wu) + bu)
    si = jax.nn.sigmoid(dot(t, wi) + bi)
    snu = jax.nn.sigmoid(dot(nt, wu) + bu)
    sni = jax.nn.sigmoid(dot(nt, wi) + bi)
    o_ref[...] = jnp.sum(u_ref[...] * (su - snu) + i_ref[...] * (si - sni), axis=1)


def _tc_dense(u, i, p, n, wut, wit, bu2, bi2):
    blk = 4096
    row_spec = pl.BlockSpec((blk, K), lambda b: (b, 0))
    full_spec = pl.BlockSpec((K, K), lambda b: (0, 0))
    bias_spec = pl.BlockSpec((1, K), lambda b: (0, 0))
    return pl.pallas_call(
        _tc_body,
        grid=(B // blk,),
        in_specs=[row_spec, row_spec, row_spec, row_spec,
                  full_spec, full_spec, bias_spec, bias_spec],
        out_specs=pl.BlockSpec((blk,), lambda b: (b,)),
        out_shape=jax.ShapeDtypeStruct((B,), jnp.float32),
    )(u, i, p, n, wut, wit, bu2, bi2)


def kernel(x, userVecs, itemVecs, tagVecs, Wu, bu, Wi, bi):
    x32 = x.astype(jnp.int32)
    uidx = x32[:, 0]
    iidx = x32[:, 1]
    pidx = x32[:, 2]
    nidx = x32[:, 3]
    u3 = userVecs.reshape(-1, TPR, K)
    i3 = itemVecs.reshape(-1, TPR, K)
    t3 = tagVecs.reshape(-1, TPR, K)
    gu, gi, gp, gn = _sc_gather4()(u3, i3, t3, uidx, iidx, pidx, nidx)
    return _tc_dense(gu, gi, gp, gn, Wu.T, Wi.T,
                     bu.reshape(1, K), bi.reshape(1, K))
